# baseline (speedup 1.0000x reference)
import functools

import jax
import jax.numpy as jnp
from jax import lax
from jax.experimental import pallas as pl
from jax.experimental.pallas import tpu as pltpu
from jax.experimental.pallas import tpu_sc as plsc

BATCH = 16384
EMBED_DIM = 32
TOTAL = BATCH * 2
NUM_NODES = 1000000
_FLAT = EMBED_DIM * NUM_NODES

_info = plsc.get_sparse_core_info()
_NC, _NS = _info.num_cores, _info.num_subcores
_NW = _NC * _NS
_PER_W = TOTAL // _NW
_CHUNK = 128
_NCHUNK = _PER_W // _CHUNK
_L = 16

_mesh = plsc.VectorSubcoreMesh(core_axis_name="c", subcore_axis_name="s")


@functools.partial(
    pl.kernel,
    mesh=_mesh,
    compiler_params=pltpu.CompilerParams(needs_layout_passes=False),
    out_type=jax.ShapeDtypeStruct((2, EMBED_DIM, BATCH), jnp.float32),
    scratch_types=[
        pltpu.VMEM((_PER_W,), jnp.int32),
        pltpu.VMEM((_NCHUNK, _CHUNK), jnp.int32),
        pltpu.VMEM((_NCHUNK, _CHUNK), jnp.int32),
        pltpu.VMEM((EMBED_DIM, _PER_W), jnp.float32),
        pltpu.SemaphoreType.DMA,
    ],
)
def _gather(idx_hbm, table_hbm, out_hbm, idx_v, offa_v, offb_v, out_v, sem):
    wid = lax.axis_index("s") * _NC + lax.axis_index("c")
    p = wid // (_NW // 2)
    jbase = (wid % (_NW // 2)) * _PER_W
    pltpu.sync_copy(idx_hbm.at[pl.ds(wid * _PER_W, _PER_W)], idx_v)

    def compute_offs(d, buf):
        base = d * NUM_NODES
        for c in range(_NCHUNK):
            for j in range(_CHUNK // _L):
                buf[c, pl.ds(j * _L, _L)] = (
                    idx_v[pl.ds(c * _CHUNK + j * _L, _L)] + base
                )

    def fire(d, buf):
        for c in range(_NCHUNK):
            pltpu.async_copy(
                table_hbm.at[buf.at[c]],
                out_v.at[d, pl.ds(c * _CHUNK, _CHUNK)],
                sem,
            )

    def drain(d):
        for c in range(_NCHUNK):
            pltpu.make_async_copy(
                table_hbm.at[offa_v.at[c]],
                out_v.at[d, pl.ds(c * _CHUNK, _CHUNK)],
                sem,
            ).wait()

    compute_offs(jnp.int32(0), offa_v)
    fire(jnp.int32(0), offa_v)

    def body(k, carry):
        d0 = 2 * k
        d1 = 2 * k + 1
        d2 = 2 * k + 2
        compute_offs(d1, offb_v)
        fire(d1, offb_v)
        drain(d0)
        compute_offs(d2, offa_v)

        @pl.when(d2 < EMBED_DIM)
        def _fire_ahead():
            fire(d2, offa_v)

        drain(d1)
        return carry

    lax.fori_loop(0, EMBED_DIM // 2, body, 0, unroll=False)

    pltpu.sync_copy(out_v, out_hbm.at[p, :, pl.ds(jbase, _PER_W)])


def kernel(node_pairs, embedding):
    idx = node_pairs.T.reshape(TOTAL)
    table_flat = embedding.T.reshape(_FLAT)
    out = _gather(idx, table_flat)
    return jnp.transpose(out, (2, 0, 1))


# speedup vs baseline: 4.8459x; 4.8459x over previous
import functools

import jax
import jax.numpy as jnp
from jax import lax
from jax.experimental import pallas as pl
from jax.experimental.pallas import tpu as pltpu
from jax.experimental.pallas import tpu_sc as plsc

BATCH = 16384
EMBED_DIM = 32
TOTAL = BATCH * 2
NUM_NODES = 1000000
_RPP = 128 // EMBED_DIM
_PAD_ROWS = NUM_NODES // _RPP

_info = plsc.get_sparse_core_info()
_NC, _NS = _info.num_cores, _info.num_subcores
_NW = _NC * _NS
_PER_W = TOTAL // _NW
_CHUNK = 128
_NCHUNK = _PER_W // _CHUNK
_L = 16

_mesh = plsc.VectorSubcoreMesh(core_axis_name="c", subcore_axis_name="s")


@functools.partial(
    pl.kernel,
    mesh=_mesh,
    compiler_params=pltpu.CompilerParams(needs_layout_passes=False),
    out_type=jax.ShapeDtypeStruct((2, EMBED_DIM, BATCH), jnp.float32),
    scratch_types=[
        pltpu.VMEM((_PER_W,), jnp.int32),
        pltpu.VMEM((_NCHUNK, _CHUNK), jnp.int32),
        pltpu.VMEM((_CHUNK, 128), jnp.float32),
        pltpu.VMEM((_CHUNK, 128), jnp.float32),
        pltpu.VMEM((EMBED_DIM, _PER_W), jnp.float32),
        pltpu.SemaphoreType.DMA,
        pltpu.SemaphoreType.DMA,
    ],
)
def _gather(idx_hbm, table_hbm, out_hbm, idx_v, g_v, pad_a, pad_b, out_v,
            sem_a, sem_b):
    wid = lax.axis_index("s") * _NC + lax.axis_index("c")
    p = wid // (_NW // 2)
    jbase = (wid % (_NW // 2)) * _PER_W
    pltpu.sync_copy(idx_hbm.at[pl.ds(wid * _PER_W, _PER_W)], idx_v)

    for c in range(_NCHUNK):
        for j in range(_CHUNK // _L):
            v = idx_v[pl.ds(c * _CHUNK + j * _L, _L)]
            g_v[c, pl.ds(j * _L, _L)] = lax.shift_right_logical(v, 2)

    bufs = (pad_a, pad_b)
    sems = (sem_a, sem_b)

    def fire(c):
        pltpu.async_copy(table_hbm.at[g_v.at[c]], bufs[c % 2], sems[c % 2])

    def drain(c):
        pltpu.make_async_copy(
            table_hbm.at[g_v.at[c]], bufs[c % 2], sems[c % 2]
        ).wait()

    fire(0)
    lanes = lax.iota(jnp.int32, _L)
    for c in range(_NCHUNK):
        if c + 1 < _NCHUNK:
            fire(c + 1)
        drain(c)
        buf = bufs[c % 2]
        for g in range(_CHUNK // _L):
            l16 = g * _L + lanes
            col16 = lax.shift_left(
                lax.bitwise_and(
                    idx_v[pl.ds(c * _CHUNK + g * _L, _L)],
                    jnp.int32(_RPP - 1),
                ),
                jnp.int32(5),
            )
            dst16 = c * _CHUNK + g * _L + lanes

            def d_body(d, carry, l16=l16, col16=col16, dst16=dst16, buf=buf):
                d16 = jnp.full((_L,), d, jnp.int32)
                vals = plsc.load_gather(buf, [l16, col16 + d])
                plsc.store_scatter(out_v, [d16, dst16], vals)
                return carry

            lax.fori_loop(0, EMBED_DIM, d_body, 0, unroll=False)

    pltpu.sync_copy(out_v, out_hbm.at[p, :, pl.ds(jbase, _PER_W)])


def kernel(node_pairs, embedding):
    idx = node_pairs.T.reshape(TOTAL)
    table = embedding.reshape(_PAD_ROWS, 128)
    out = _gather(idx, table)
    return jnp.transpose(out, (2, 0, 1))
